# Initial kernel scaffold; baseline (speedup 1.0000x reference)
import jax
import jax.numpy as jnp
from jax.experimental import pallas as pl


def kernel(t, u, A_W, A_b, Fc_W, Fc_b, Fh_W, Fh_b, Gc_W1, Gc_b1, Gc_W2, Gc_b2, Gc_W3, Gc_b3):
    raise NotImplementedError("write your pallas kernel here")


# speedup vs baseline: 1.7069x; 1.7069x over previous
import jax
import jax.numpy as jnp
from jax.experimental import pallas as pl
from jax.experimental.pallas import tpu as pltpu

P = 1024
Q = 1024
NH = 512
DECAY = 0.001
TB = 512


def _softplus(x):
    return jnp.maximum(x, 0.0) + jnp.log(1.0 + jnp.exp(-jnp.abs(x)))


def _celu(x):
    return jnp.where(x > 0.0, x, jnp.exp(jnp.minimum(x, 0.0)) - 1.0)


def _odefunc_kernel(u_ref, aw_ref, ab_ref, wcat_ref, bcat_ref,
                    g2w_ref, g2b_ref, g3w_ref, g3b_ref, out_ref):
    u = u_ref[...]
    c = u[:, :P]
    h = u[:, P:]

    h_ = _softplus(
        jnp.dot(h, aw_ref[...], preferred_element_type=jnp.float32)
        + ab_ref[...])

    y = (jnp.dot(c, wcat_ref[:P, :], preferred_element_type=jnp.float32)
         + jnp.dot(h_, wcat_ref[P:, :], preferred_element_type=jnp.float32)
         + bcat_ref[...])
    fc = _softplus(y[:, :P])
    fh = _softplus(y[:, P:P + Q])
    g = _celu(y[:, P + Q:])

    g = _celu(
        jnp.dot(g, g2w_ref[...], preferred_element_type=jnp.float32)
        + g2b_ref[...])
    g = (jnp.dot(g, g3w_ref[...], preferred_element_type=jnp.float32)
         + g3b_ref[...])

    out_ref[:, :P] = -fc * c + g - DECAY * c
    out_ref[:, P:] = -fh * h


def kernel(t, u, A_W, A_b, Fc_W, Fc_b, Fh_W, Fh_b,
           Gc_W1, Gc_b1, Gc_W2, Gc_b2, Gc_W3, Gc_b3):
    B = u.shape[0]
    u2 = u.reshape(B, P + Q)

    aw = A_W[:, :Q].T
    wcat = jnp.concatenate([Fc_W.T, Fh_W.T, Gc_W1.T], axis=1)
    bcat = jnp.concatenate([Fc_b, Fh_b, Gc_b1]).reshape(1, -1)
    g2w = Gc_W2.T
    g3w = Gc_W3.T

    grid = (B // TB,)
    out = pl.pallas_call(
        _odefunc_kernel,
        grid=grid,
        in_specs=[
            pl.BlockSpec((TB, P + Q), lambda i: (i, 0)),
            pl.BlockSpec((Q, Q), lambda i: (0, 0)),
            pl.BlockSpec((1, Q), lambda i: (0, 0)),
            pl.BlockSpec((P + Q, P + Q + NH), lambda i: (0, 0)),
            pl.BlockSpec((1, P + Q + NH), lambda i: (0, 0)),
            pl.BlockSpec((NH, NH), lambda i: (0, 0)),
            pl.BlockSpec((1, NH), lambda i: (0, 0)),
            pl.BlockSpec((NH, P), lambda i: (0, 0)),
            pl.BlockSpec((1, P), lambda i: (0, 0)),
        ],
        out_specs=pl.BlockSpec((TB, P + Q), lambda i: (i, 0)),
        out_shape=jax.ShapeDtypeStruct((B, P + Q), jnp.float32),
        compiler_params=pltpu.CompilerParams(
            dimension_semantics=("arbitrary",),
        ),
    )(u2, aw, A_b.reshape(1, Q), wcat, bcat,
      g2w, Gc_b2.reshape(1, NH), g3w, Gc_b3.reshape(1, P))

    return out.reshape(B, 1, P + Q)


# speedup vs baseline: 1.8335x; 1.0742x over previous
import jax
import jax.numpy as jnp
from jax.experimental import pallas as pl
from jax.experimental.pallas import tpu as pltpu

P = 1024
Q = 1024
NH = 512
DECAY = 0.001
TB = 512


def _softplus(x):
    return jnp.maximum(x, 0.0) + jnp.log(1.0 + jnp.exp(-jnp.abs(x)))


def _celu(x):
    return jnp.where(x > 0.0, x, jnp.exp(jnp.minimum(x, 0.0)) - 1.0)


def _odefunc_kernel(u_ref, aw_ref, ab_ref, wcat_ref, bcat_ref,
                    g2w_ref, g2b_ref, g3w_ref, g3b_ref, out_ref):
    u = u_ref[...]
    c = u[:, :P]
    h = u[:, P:]

    bf16 = jnp.bfloat16

    h_ = _softplus(
        jnp.dot(h.astype(bf16), aw_ref[...], preferred_element_type=jnp.float32)
        + ab_ref[...])

    y = (jnp.dot(c.astype(bf16), wcat_ref[:P, :],
                 preferred_element_type=jnp.float32)
         + jnp.dot(h_.astype(bf16), wcat_ref[P:, :],
                   preferred_element_type=jnp.float32)
         + bcat_ref[...])
    fc = _softplus(y[:, :P])
    fh = _softplus(y[:, P:P + Q])
    g = _celu(y[:, P + Q:])

    g = _celu(
        jnp.dot(g.astype(bf16), g2w_ref[...],
                preferred_element_type=jnp.float32)
        + g2b_ref[...])
    g = (jnp.dot(g.astype(bf16), g3w_ref[...],
                 preferred_element_type=jnp.float32)
         + g3b_ref[...])

    out_ref[:, :P] = -fc * c + g - DECAY * c
    out_ref[:, P:] = -fh * h


def kernel(t, u, A_W, A_b, Fc_W, Fc_b, Fh_W, Fh_b,
           Gc_W1, Gc_b1, Gc_W2, Gc_b2, Gc_W3, Gc_b3):
    B = u.shape[0]
    u2 = u.reshape(B, P + Q)

    bf16 = jnp.bfloat16
    aw = A_W[:, :Q].T.astype(bf16)
    wcat = jnp.concatenate([Fc_W.T, Fh_W.T, Gc_W1.T],
                           axis=1).astype(bf16)
    bcat = jnp.concatenate([Fc_b, Fh_b, Gc_b1]).reshape(1, -1)
    g2w = Gc_W2.T.astype(bf16)
    g3w = Gc_W3.T.astype(bf16)

    grid = (B // TB,)
    out = pl.pallas_call(
        _odefunc_kernel,
        grid=grid,
        in_specs=[
            pl.BlockSpec((TB, P + Q), lambda i: (i, 0)),
            pl.BlockSpec((Q, Q), lambda i: (0, 0)),
            pl.BlockSpec((1, Q), lambda i: (0, 0)),
            pl.BlockSpec((P + Q, P + Q + NH), lambda i: (0, 0)),
            pl.BlockSpec((1, P + Q + NH), lambda i: (0, 0)),
            pl.BlockSpec((NH, NH), lambda i: (0, 0)),
            pl.BlockSpec((1, NH), lambda i: (0, 0)),
            pl.BlockSpec((NH, P), lambda i: (0, 0)),
            pl.BlockSpec((1, P), lambda i: (0, 0)),
        ],
        out_specs=pl.BlockSpec((TB, P + Q), lambda i: (i, 0)),
        out_shape=jax.ShapeDtypeStruct((B, P + Q), jnp.float32),
        compiler_params=pltpu.CompilerParams(
            dimension_semantics=("arbitrary",),
        ),
    )(u2, aw, A_b.reshape(1, Q), wcat, bcat,
      g2w, Gc_b2.reshape(1, NH), g3w, Gc_b3.reshape(1, P))

    return out.reshape(B, 1, P + Q)


# speedup vs baseline: 1.9241x; 1.0494x over previous
import jax
import jax.numpy as jnp
from jax import lax
from jax.experimental import pallas as pl
from jax.experimental.pallas import tpu as pltpu

P = 1024
Q = 1024
NH = 512
DECAY = 0.001
TB = 512

_DNT = (((1,), (1,)), ((), ()))


def _dott(x, w):
    return lax.dot_general(x, w, _DNT, preferred_element_type=jnp.float32)


def _softplus(x):
    return jnp.maximum(x, 0.0) + jnp.log(1.0 + jnp.exp(-jnp.abs(x)))


def _celu(x):
    return jnp.where(x > 0.0, x, jnp.exp(jnp.minimum(x, 0.0)) - 1.0)


def _odefunc_kernel(u_ref, aw_ref, ab_ref, fcw_ref, fcb_ref, fhw_ref, fhb_ref,
                    g1w_ref, g1b_ref, g2w_ref, g2b_ref, g3w_ref, g3b_ref,
                    out_ref):
    bf16 = jnp.bfloat16
    u = u_ref[...]
    c = u[:, :P]
    h = u[:, P:]
    cb = c.astype(bf16)
    hb = h.astype(bf16)

    h_ = _softplus(_dott(hb, aw_ref[...]) + ab_ref[...])
    hb_ = h_.astype(bf16)

    fc = _softplus(_dott(cb, fcw_ref[:, :P]) + _dott(hb_, fcw_ref[:, P:])
                   + fcb_ref[...])
    fh = _softplus(_dott(cb, fhw_ref[:, :P]) + _dott(hb_, fhw_ref[:, P:])
                   + fhb_ref[...])
    g = _celu(_dott(cb, g1w_ref[:, :P]) + _dott(hb_, g1w_ref[:, P:])
              + g1b_ref[...])
    g = _celu(_dott(g.astype(bf16), g2w_ref[...]) + g2b_ref[...])
    g = _dott(g.astype(bf16), g3w_ref[...]) + g3b_ref[...]

    out_ref[:, :P] = -fc * c + g - DECAY * c
    out_ref[:, P:] = -fh * h


def kernel(t, u, A_W, A_b, Fc_W, Fc_b, Fh_W, Fh_b,
           Gc_W1, Gc_b1, Gc_W2, Gc_b2, Gc_W3, Gc_b3):
    B = u.shape[0]
    u2 = u.reshape(B, P + Q)

    bf16 = jnp.bfloat16
    aw = A_W[:, :Q].astype(bf16)
    fcw = Fc_W.astype(bf16)
    fhw = Fh_W.astype(bf16)
    g1w = Gc_W1.astype(bf16)
    g2w = Gc_W2.astype(bf16)
    g3w = Gc_W3.astype(bf16)

    grid = (B // TB,)
    inv = lambda i: (0, 0)
    out = pl.pallas_call(
        _odefunc_kernel,
        grid=grid,
        in_specs=[
            pl.BlockSpec((TB, P + Q), lambda i: (i, 0)),
            pl.BlockSpec((Q, Q), inv),
            pl.BlockSpec((1, Q), inv),
            pl.BlockSpec((P, P + Q), inv),
            pl.BlockSpec((1, P), inv),
            pl.BlockSpec((Q, P + Q), inv),
            pl.BlockSpec((1, Q), inv),
            pl.BlockSpec((NH, P + Q), inv),
            pl.BlockSpec((1, NH), inv),
            pl.BlockSpec((NH, NH), inv),
            pl.BlockSpec((1, NH), inv),
            pl.BlockSpec((P, NH), inv),
            pl.BlockSpec((1, P), inv),
        ],
        out_specs=pl.BlockSpec((TB, P + Q), lambda i: (i, 0)),
        out_shape=jax.ShapeDtypeStruct((B, P + Q), jnp.float32),
        compiler_params=pltpu.CompilerParams(
            dimension_semantics=("arbitrary",),
        ),
    )(u2, aw, A_b.reshape(1, Q), fcw, Fc_b.reshape(1, P),
      fhw, Fh_b.reshape(1, Q), g1w, Gc_b1.reshape(1, NH),
      g2w, Gc_b2.reshape(1, NH), g3w, Gc_b3.reshape(1, P))

    return out.reshape(B, 1, P + Q)


# speedup vs baseline: 2.2280x; 1.1579x over previous
import jax
import jax.numpy as jnp
from jax import lax
from jax.experimental import pallas as pl
from jax.experimental.pallas import tpu as pltpu

P = 1024
Q = 1024
NH = 512
DECAY = 0.001
TB = 512

_DNT = (((1,), (1,)), ((), ()))


def _dott(x, w):
    return lax.dot_general(x, w, _DNT, preferred_element_type=jnp.float32)


def _softplus(x):
    return jnp.maximum(x, 0.0) + jnp.log(1.0 + jnp.exp(-jnp.abs(x)))


def _celu(x):
    return jnp.where(x > 0.0, x, jnp.exp(jnp.minimum(x, 0.0)) - 1.0)


def _odefunc_kernel(u_ref, aw_ref, ab_ref, fcw_ref, fcb_ref, fhw_ref, fhb_ref,
                    g1w_ref, g1b_ref, g2w_ref, g2b_ref, g3w_ref, g3b_ref,
                    out_ref):
    bf16 = jnp.bfloat16
    u = u_ref[...]
    c = u[:, :P]
    h = u[:, P:]
    cb = c.astype(bf16)
    hb = h.astype(bf16)

    h_ = _softplus(_dott(hb, aw_ref[...]) + ab_ref[...])
    hb_ = h_.astype(bf16)

    fc = _softplus(_dott(cb, fcw_ref[:, :P]) + _dott(hb_, fcw_ref[:, P:])
                   + fcb_ref[...])
    fh = _softplus(_dott(cb, fhw_ref[:, :P]) + _dott(hb_, fhw_ref[:, P:])
                   + fhb_ref[...])
    g = _celu(_dott(cb, g1w_ref[:, :P]) + _dott(hb_, g1w_ref[:, P:])
              + g1b_ref[...])
    g = _celu(_dott(g.astype(bf16), g2w_ref[...]) + g2b_ref[...])
    g = _dott(g.astype(bf16), g3w_ref[...]) + g3b_ref[...]

    out_ref[:, :P] = -fc * c + g - DECAY * c
    out_ref[:, P:] = -fh * h


def kernel(t, u, A_W, A_b, Fc_W, Fc_b, Fh_W, Fh_b,
           Gc_W1, Gc_b1, Gc_W2, Gc_b2, Gc_W3, Gc_b3):
    B = u.shape[0]

    bf16 = jnp.bfloat16
    aw = A_W.astype(bf16)
    fcw = Fc_W.astype(bf16)
    fhw = Fh_W.astype(bf16)
    g1w = Gc_W1.astype(bf16)
    g2w = Gc_W2.astype(bf16)
    g3w = Gc_W3.astype(bf16)

    grid = (B // TB,)
    inv = lambda i: (0, 0)
    out = pl.pallas_call(
        _odefunc_kernel,
        grid=grid,
        in_specs=[
            pl.BlockSpec((TB, None, P + Q), lambda i: (i, 0, 0)),
            pl.BlockSpec((Q, Q), inv),
            pl.BlockSpec((1, Q), inv),
            pl.BlockSpec((P, P + Q), inv),
            pl.BlockSpec((1, P), inv),
            pl.BlockSpec((Q, P + Q), inv),
            pl.BlockSpec((1, Q), inv),
            pl.BlockSpec((NH, P + Q), inv),
            pl.BlockSpec((1, NH), inv),
            pl.BlockSpec((NH, NH), inv),
            pl.BlockSpec((1, NH), inv),
            pl.BlockSpec((P, NH), inv),
            pl.BlockSpec((1, P), inv),
        ],
        out_specs=pl.BlockSpec((TB, None, P + Q), lambda i: (i, 0, 0)),
        out_shape=jax.ShapeDtypeStruct((B, 1, P + Q), jnp.float32),
        compiler_params=pltpu.CompilerParams(
            dimension_semantics=("arbitrary",),
        ),
    )(u, aw, A_b.reshape(1, Q), fcw, Fc_b.reshape(1, P),
      fhw, Fh_b.reshape(1, Q), g1w, Gc_b1.reshape(1, NH),
      g2w, Gc_b2.reshape(1, NH), g3w, Gc_b3.reshape(1, P))

    return out
